# baseline (speedup 1.0000x reference)
import functools

import jax
import jax.numpy as jnp
from jax import lax
from jax.experimental import pallas as pl
from jax.experimental.pallas import tpu as pltpu
from jax.experimental.pallas import tpu_sc as plsc

NQ = 100000
NH = 100000
NS = 16
C = 32

NQP = 102400
STEPQ = 1024
TOT_STEPS = NQP // STEPQ
FAST_STEPS = 76
NSTREAM = STEPQ // 128

_DN_T = (((0,), (0,)), ((), ()))


NH_PAD = 102400
_PREP_B = 6400
_PS = _PREP_B // 4


def _prep_body(xyzT_ref, featT_ref, w1a_ref, w1b_ref, b_ref, a_ref):
    a_rows = (
        lax.dot_general(xyzT_ref[...], w1a_ref[...], _DN_T,
                        preferred_element_type=jnp.float32)
        + lax.dot_general(featT_ref[...], w1b_ref[...], _DN_T,
                          preferred_element_type=jnp.float32)
    )
    a_ref[...] = jnp.concatenate(
        [a_rows[0:_PS], a_rows[_PS:2 * _PS],
         a_rows[2 * _PS:3 * _PS], a_rows[3 * _PS:4 * _PS]], axis=1
    ) + b_ref[...]


def _prep_call(xyzT, featT, w1a, w1b, b1t):
    grid = NH_PAD // _PREP_B
    return pl.pallas_call(
        _prep_body,
        grid=(grid,),
        in_specs=[
            pl.BlockSpec((3, _PREP_B), lambda i: (0, i)),
            pl.BlockSpec((C, _PREP_B), lambda i: (0, i)),
            pl.BlockSpec((3, C), lambda i: (0, 0)),
            pl.BlockSpec((C, C), lambda i: (0, 0)),
            pl.BlockSpec((1, 128), lambda i: (0, 0)),
        ],
        out_specs=pl.BlockSpec((_PREP_B // 4, 128), lambda i: (i, 0)),
        out_shape=jax.ShapeDtypeStruct((NH_PAD // 4, 128), jnp.float32),
    )(xyzT, featT, w1a, w1b, b1t)


@functools.cache
def _sc_gather_kernel():
    @functools.partial(
        pl.kernel,
        out_type=jax.ShapeDtypeStruct((NS * NQP, C), jnp.float32),
        mesh=plsc.VectorSubcoreMesh(
            core_axis_name="c", subcore_axis_name="s",
            num_cores=2, num_subcores=16),
        scratch_types=[
            pltpu.VMEM((NSTREAM, 128), jnp.int32),
            pltpu.VMEM((NSTREAM, 128), jnp.int32),
            pltpu.VMEM((STEPQ, C), jnp.float32),
            pltpu.VMEM((STEPQ, C), jnp.float32),
            pltpu.SemaphoreType.DMA,
            pltpu.SemaphoreType.DMA,
            pltpu.SemaphoreType.DMA,
        ],
        compiler_params=pltpu.CompilerParams(use_tc_tiling_on_sc=False),
    )
    def _sc_gather(table_hbm, idxt_hbm, out_hbm,
                   iv0, iv1, rv0, rv1, sem0, sem1, sem_w):
        core = lax.axis_index("c")
        plane = lax.axis_index("s")
        qbase = plane * NQP

        def load_idx(t, iv):
            r0 = t * NSTREAM
            pltpu.sync_copy(idxt_hbm.at[plane, pl.ds(r0, NSTREAM)], iv)

        def fire(iv, rv, sem):
            return [
                pltpu.async_copy(
                    table_hbm.at[iv.at[k]],
                    rv.at[pl.ds(k * 128, 128)],
                    sem,
                )
                for k in range(NSTREAM)
            ]

        def wb(t, rv):
            q0 = qbase + t * STEPQ
            return pltpu.async_copy(rv, out_hbm.at[pl.ds(q0, STEPQ)], sem_w)

        def step(tt, carry):
            t0 = tt * 2 + core
            load_idx(t0, iv0)
            g0 = fire(iv0, rv0, sem0)
            for cp in g0:
                cp.wait()
            pltpu.sync_copy(rv0, out_hbm.at[pl.ds(qbase + t0 * STEPQ, STEPQ)])
            return carry

        lax.fori_loop(0, TOT_STEPS // 2, step, 0)

    return _sc_gather


_B = 4096


def _mlp_body(g_ref, lxT_ref, lfT_ref, w1a_ref, eye_ref, bdw2_ref, b2_ref,
              out_ref):
    b4 = _B // 4
    q_rows = lax.dot_general(lxT_ref[...], w1a_ref[...], _DN_T,
                             preferred_element_type=jnp.float32)
    qid = pl.program_id(0) * _B + lax.broadcasted_iota(jnp.int32, (_B, C), 0)
    q_rows = jnp.where(qid < NQ, q_rows, 0.0)
    q4 = jnp.concatenate(
        [q_rows[0:b4], q_rows[b4:2 * b4],
         q_rows[2 * b4:3 * b4], q_rows[3 * b4:4 * b4]], axis=1)
    w2 = bdw2_ref[...]
    b2 = b2_ref[...]
    acc = None
    for j in range(NS):
        h = jnp.maximum(g_ref[j] - q4, 0.0)
        y = jnp.maximum(
            jnp.dot(h, w2, preferred_element_type=jnp.float32) + b2, 0.0)
        acc = y if acc is None else jnp.maximum(acc, y)
    m_rows = jnp.concatenate(
        [acc[:, 0:C], acc[:, C:2 * C],
         acc[:, 2 * C:3 * C], acc[:, 3 * C:4 * C]], axis=0)
    lf_rows = lax.dot_general(lfT_ref[...], eye_ref[...], _DN_T,
                              preferred_element_type=jnp.float32)
    out_ref[...] = jnp.concatenate([lf_rows, m_rows], axis=1)


def _mlp_call(g3, lxT, lfT, w1a, eye32, bdw2, b2t):
    grid = (NQ + _B - 1) // _B
    b4 = _B // 4
    return pl.pallas_call(
        _mlp_body,
        grid=(grid,),
        in_specs=[
            pl.BlockSpec((NS, b4, 128), lambda i: (0, i, 0)),
            pl.BlockSpec((3, _B), lambda i: (0, i)),
            pl.BlockSpec((C, _B), lambda i: (0, i)),
            pl.BlockSpec((3, C), lambda i: (0, 0)),
            pl.BlockSpec((C, C), lambda i: (0, 0)),
            pl.BlockSpec((128, 128), lambda i: (0, 0)),
            pl.BlockSpec((1, 128), lambda i: (0, 0)),
        ],
        out_specs=pl.BlockSpec((_B, 2 * C), lambda i: (i, 0)),
        out_shape=jax.ShapeDtypeStruct((NQ, 2 * C), jnp.float32),
    )(g3, lxT, lfT, w1a, eye32, bdw2, b2t)


def kernel(lr_xyz, hr_xyz, lr_feat, hr_feat, neigh_idx, W1, b1, W2, b2):
    w1a = W1[:3]
    w1b = W1[3:]
    eye4 = jnp.eye(4, dtype=jnp.float32)
    eye32 = jnp.eye(C, dtype=jnp.float32)
    bdw2 = jnp.kron(eye4, W2)
    b1t = jnp.tile(b1, 4)[None, :]
    b2t = jnp.tile(b2, 4)[None, :]

    a4 = _prep_call(hr_xyz.T, hr_feat.T, w1a, w1b, b1t)
    table = a4.reshape(NH_PAD, C)

    v = neigh_idx.astype(jnp.int32).T
    loc = v % _PREP_B
    vrow = (v - loc) + 4 * (loc % _PS) + loc // _PS

    idxt = jnp.pad(vrow, ((0, 0), (0, NQP - NQ)))
    idxt = idxt.reshape(NS, NQP // _B, 4, _B // 4).transpose(0, 1, 3, 2)
    idxt = idxt.reshape(NS, NQP // 128, 128)

    g = _sc_gather_kernel()(table, idxt)
    g3 = g.reshape(NS, NQP // 4, 128)

    return _mlp_call(g3, lr_xyz.T, lr_feat.T, w1a, eye32, bdw2, b2t)


# speedup vs baseline: 1.0578x; 1.0578x over previous
import functools

import jax
import jax.numpy as jnp
from jax import lax
from jax.experimental import pallas as pl
from jax.experimental.pallas import tpu as pltpu
from jax.experimental.pallas import tpu_sc as plsc

NQ = 100000
NH = 100000
NS = 16
C = 32

NQP = 102400
STEPQ = 1024
TOT_STEPS = NQP // STEPQ
NSTREAM = STEPQ // 128

_DN_T = (((0,), (0,)), ((), ()))


NH_PAD = 102400
_PREP_B = 6400
_PS = _PREP_B // 4


def _prep_body(xyzT_ref, featT_ref, w1a_ref, w1b_ref, b_ref, a_ref):
    a_rows = (
        lax.dot_general(xyzT_ref[...], w1a_ref[...], _DN_T,
                        preferred_element_type=jnp.float32)
        + lax.dot_general(featT_ref[...], w1b_ref[...], _DN_T,
                          preferred_element_type=jnp.float32)
    )
    a_ref[...] = jnp.concatenate(
        [a_rows[0:_PS], a_rows[_PS:2 * _PS],
         a_rows[2 * _PS:3 * _PS], a_rows[3 * _PS:4 * _PS]], axis=1
    ) + b_ref[...]


def _prep_call(xyzT, featT, w1a, w1b, b1t):
    grid = NH_PAD // _PREP_B
    return pl.pallas_call(
        _prep_body,
        grid=(grid,),
        in_specs=[
            pl.BlockSpec((3, _PREP_B), lambda i: (0, i)),
            pl.BlockSpec((C, _PREP_B), lambda i: (0, i)),
            pl.BlockSpec((3, C), lambda i: (0, 0)),
            pl.BlockSpec((C, C), lambda i: (0, 0)),
            pl.BlockSpec((1, 128), lambda i: (0, 0)),
        ],
        out_specs=pl.BlockSpec((_PREP_B // 4, 128), lambda i: (i, 0)),
        out_shape=jax.ShapeDtypeStruct((NH_PAD // 4, 128), jnp.float32),
    )(xyzT, featT, w1a, w1b, b1t)


@functools.cache
def _sc_gather_kernel():
    @functools.partial(
        pl.kernel,
        out_type=jax.ShapeDtypeStruct((NS * NQP, C), jnp.float32),
        mesh=plsc.VectorSubcoreMesh(
            core_axis_name="c", subcore_axis_name="s",
            num_cores=2, num_subcores=16),
        scratch_types=[
            pltpu.VMEM((NSTREAM, 128), jnp.int32),
            pltpu.VMEM((NSTREAM, 128), jnp.int32),
            pltpu.VMEM((STEPQ, C), jnp.float32),
            pltpu.VMEM((STEPQ, C), jnp.float32),
            pltpu.SemaphoreType.DMA,
            pltpu.SemaphoreType.DMA,
            pltpu.SemaphoreType.DMA,
        ],
        compiler_params=pltpu.CompilerParams(use_tc_tiling_on_sc=False),
    )
    def _sc_gather(table_hbm, idxt_hbm, out_hbm,
                   iv0, iv1, rv0, rv1, sem0, sem1, sem_w):
        core = lax.axis_index("c")
        plane = lax.axis_index("s")
        qbase = plane * NQP

        def load_idx(t, iv):
            r0 = t * NSTREAM
            pltpu.sync_copy(idxt_hbm.at[plane, pl.ds(r0, NSTREAM)], iv)

        def fire(iv, rv, sem):
            return [
                pltpu.async_copy(
                    table_hbm.at[iv.at[k]],
                    rv.at[pl.ds(k * 128, 128)],
                    sem,
                )
                for k in range(NSTREAM)
            ]

        def wb(t, rv):
            q0 = qbase + t * STEPQ
            return pltpu.async_copy(rv, out_hbm.at[pl.ds(q0, STEPQ)], sem_w)

        def pair(tt, carry):
            t0 = tt * 4 + core
            t1 = t0 + 2
            load_idx(t0, iv0)
            g0 = fire(iv0, rv0, sem0)
            load_idx(t1, iv1)
            g1 = fire(iv1, rv1, sem1)
            for cp in g0:
                cp.wait()
            w0 = wb(t0, rv0)
            for cp in g1:
                cp.wait()
            w1 = wb(t1, rv1)
            w0.wait()
            w1.wait()
            return carry

        lax.fori_loop(0, TOT_STEPS // 4, pair, 0)

    return _sc_gather


_B = 4096


def _mlp_body(g_ref, lxT_ref, lfT_ref, w1a_ref, eye_ref, bdw2_ref, b2_ref,
              out_ref):
    b4 = _B // 4
    q_rows = lax.dot_general(lxT_ref[...], w1a_ref[...], _DN_T,
                             preferred_element_type=jnp.float32)
    qid = pl.program_id(0) * _B + lax.broadcasted_iota(jnp.int32, (_B, C), 0)
    q_rows = jnp.where(qid < NQ, q_rows, 0.0)
    q4 = jnp.concatenate(
        [q_rows[0:b4], q_rows[b4:2 * b4],
         q_rows[2 * b4:3 * b4], q_rows[3 * b4:4 * b4]], axis=1)
    w2 = bdw2_ref[...]
    b2 = b2_ref[...]
    acc = None
    for j in range(NS):
        h = jnp.maximum(g_ref[j] - q4, 0.0)
        y = jnp.maximum(
            jnp.dot(h, w2, preferred_element_type=jnp.float32) + b2, 0.0)
        acc = y if acc is None else jnp.maximum(acc, y)
    m_rows = jnp.concatenate(
        [acc[:, 0:C], acc[:, C:2 * C],
         acc[:, 2 * C:3 * C], acc[:, 3 * C:4 * C]], axis=0)
    lf_rows = lax.dot_general(lfT_ref[...], eye_ref[...], _DN_T,
                              preferred_element_type=jnp.float32)
    out_ref[...] = jnp.concatenate([lf_rows, m_rows], axis=1)


def _mlp_call(g3, lxT, lfT, w1a, eye32, bdw2, b2t):
    grid = (NQ + _B - 1) // _B
    b4 = _B // 4
    return pl.pallas_call(
        _mlp_body,
        grid=(grid,),
        in_specs=[
            pl.BlockSpec((NS, b4, 128), lambda i: (0, i, 0)),
            pl.BlockSpec((3, _B), lambda i: (0, i)),
            pl.BlockSpec((C, _B), lambda i: (0, i)),
            pl.BlockSpec((3, C), lambda i: (0, 0)),
            pl.BlockSpec((C, C), lambda i: (0, 0)),
            pl.BlockSpec((128, 128), lambda i: (0, 0)),
            pl.BlockSpec((1, 128), lambda i: (0, 0)),
        ],
        out_specs=pl.BlockSpec((_B, 2 * C), lambda i: (i, 0)),
        out_shape=jax.ShapeDtypeStruct((NQ, 2 * C), jnp.float32),
    )(g3, lxT, lfT, w1a, eye32, bdw2, b2t)


def kernel(lr_xyz, hr_xyz, lr_feat, hr_feat, neigh_idx, W1, b1, W2, b2):
    w1a = W1[:3]
    w1b = W1[3:]
    eye4 = jnp.eye(4, dtype=jnp.float32)
    eye32 = jnp.eye(C, dtype=jnp.float32)
    bdw2 = jnp.kron(eye4, W2)
    b1t = jnp.tile(b1, 4)[None, :]
    b2t = jnp.tile(b2, 4)[None, :]

    a4 = _prep_call(hr_xyz.T, hr_feat.T, w1a, w1b, b1t)
    table = a4.reshape(NH_PAD, C)

    v = neigh_idx.astype(jnp.int32).T
    loc = v % _PREP_B
    vrow = (v - loc) + 4 * (loc % _PS) + loc // _PS

    idxt = jnp.pad(vrow, ((0, 0), (0, NQP - NQ)))
    idxt = idxt.reshape(NS, NQP // _B, 4, _B // 4).transpose(0, 1, 3, 2)
    idxt = idxt.reshape(NS, NQP // 128, 128)

    g = _sc_gather_kernel()(table, idxt)
    g3 = g.reshape(NS, NQP // 4, 128)

    return _mlp_call(g3, lr_xyz.T, lr_feat.T, w1a, eye32, bdw2, b2t)
